# baseline (speedup 1.0000x reference)
import functools

import jax
import jax.numpy as jnp
from jax import lax
from jax.experimental import pallas as pl
from jax.experimental.pallas import tpu as pltpu
from jax.experimental.pallas import tpu_sc as plsc

B = 16384
NUM_DENSE = 13
F = 26
V = 100000
D = 32

_LEAK = 0.01


_V4 = V // 4


def _repack_body(t_ref, ident_ref, out_hbm, obuf, osem):
    f = pl.program_id(0)
    ident = ident_ref[...]
    x = t_ref[0]
    x4 = jnp.concatenate(
        [x[:, a * _V4:(a + 1) * _V4] for a in range(4)], axis=0)
    @pl.when(f > 0)
    def _():
        pltpu.make_async_copy(
            obuf, out_hbm.at[pl.ds((f - 1) * _V4, _V4)], osem).wait()
    obuf[...] = jax.lax.dot_general(
        x4, ident, (((0,), (0,)), ((), ())),
        preferred_element_type=jnp.float32)
    pltpu.make_async_copy(
        obuf, out_hbm.at[pl.ds(f * _V4, _V4)], osem).start()

    @pl.when(f == F - 1)
    def _():
        pltpu.make_async_copy(
            obuf, out_hbm.at[pl.ds(f * _V4, _V4)], osem).wait()


@functools.cache
def _make_repack():
    return pl.pallas_call(
        _repack_body,
        grid=(F,),
        in_specs=[
            pl.BlockSpec((1, D, V), lambda f: (f, 0, 0)),
            pl.BlockSpec((128, 128), lambda f: (0, 0)),
        ],
        out_specs=pl.BlockSpec(memory_space=pl.ANY),
        out_shape=jax.ShapeDtypeStruct((F * V * D // 128, 128), jnp.float32),
        scratch_shapes=[
            pltpu.VMEM((_V4, 128), jnp.float32),
            pltpu.SemaphoreType.DMA,
        ],
        compiler_params=pltpu.CompilerParams(
            vmem_limit_bytes=100 * 1024 * 1024),
    )


_CHUNK = 1024
_RING = 2


@functools.cache
def _make_sc_gather():
    info = plsc.get_sparse_core_info()
    nw = info.num_cores * info.num_subcores
    n_rows = B * F
    rows_per_w = n_rows // nw
    n_chunks = rows_per_w // _CHUNK
    assert n_chunks * _CHUNK == rows_per_w

    mesh = plsc.VectorSubcoreMesh(core_axis_name="c", subcore_axis_name="s")

    @functools.partial(
        pl.kernel,
        out_type=jax.ShapeDtypeStruct((n_rows, D), jnp.float32),
        mesh=mesh,
        scratch_types=[
            pltpu.VMEM((rows_per_w,), jnp.int32),
        ] + [pltpu.VMEM((_CHUNK, D), jnp.float32) for _ in range(_RING)]
          + [pltpu.SemaphoreType.DMA for _ in range(2 * _RING)],
        compiler_params=pltpu.CompilerParams(use_tc_tiling_on_sc=False),
    )
    def gather_kernel(tab2d, idx_hbm, out2d, idx_v, *rest):
        bufs = rest[:_RING]
        gsems = rest[_RING:2 * _RING]
        osems = rest[2 * _RING:3 * _RING]
        wid = lax.axis_index("s") * info.num_cores + lax.axis_index("c")
        base = wid * rows_per_w
        pltpu.sync_copy(idx_hbm.at[pl.ds(base, rows_per_w)], idx_v)

        wb = [None] * _RING
        for c in range(n_chunks):
            p = c % _RING
            if wb[p] is not None:
                wb[p].wait()
            g = pltpu.async_copy(
                tab2d.at[idx_v.at[pl.ds(c * _CHUNK, _CHUNK)]],
                bufs[p], gsems[p])
            g.wait()
            wb[p] = pltpu.async_copy(
                bufs[p], out2d.at[pl.ds(base + c * _CHUNK, _CHUNK)], osems[p])
        for w in wb:
            if w is not None:
                w.wait()

    return gather_kernel


def _leaky(x):
    return jnp.where(x >= 0, x, _LEAK * x)


def _split_w(w):
    wh = w.astype(jnp.bfloat16)
    wl = (w - wh.astype(jnp.float32)).astype(jnp.bfloat16)
    return wh, wl


def _dot3(a, wh_ref, wl_ref):
    ah = a.astype(jnp.bfloat16)
    al = (a - ah.astype(jnp.float32)).astype(jnp.bfloat16)
    d = functools.partial(jnp.dot, preferred_element_type=jnp.float32)
    wh, wl = wh_ref[...], wl_ref[...]
    return d(ah, wh) + d(ah, wl) + d(al, wh)


def _bot_body(num_ref, bw0h, bw0l, bb0, bw1h, bw1l, bb1, bw2h, bw2l, bb2,
              out_ref):
    x = num_ref[...]
    x = _leaky(_dot3(x, bw0h, bw0l) + bb0[...])
    x = _leaky(_dot3(x, bw1h, bw1l) + bb1[...])
    out_ref[...] = _leaky(_dot3(x, bw2h, bw2l) + bb2[...])


def _top_body(x_ref, emb_ref, tw0ah, tw0al, tw0bh, tw0bl, tb0,
              tw1h, tw1l, tb1, tw2h, tw2l, tb2, out_ref):
    x = x_ref[...]
    e = emb_ref[...]
    h = _leaky(_dot3(x, tw0ah, tw0al) + _dot3(e, tw0bh, tw0bl) + tb0[...])
    h = _leaky(_dot3(h, tw1h, tw1l) + tb1[...])
    out_ref[...] = jax.nn.sigmoid(_dot3(h, tw2h, tw2l) + tb2[...])


def _row_block(i):
    return (i, 0)


def _whole(i):
    return (0, 0)


def _w2(shape):
    return [pl.BlockSpec(shape, _whole), pl.BlockSpec(shape, _whole)]


@functools.cache
def _make_bot(bb: int):
    return pl.pallas_call(
        _bot_body,
        grid=(B // bb,),
        in_specs=[
            pl.BlockSpec((bb, NUM_DENSE), _row_block),
            *_w2((NUM_DENSE, 512)),
            pl.BlockSpec((1, 512), _whole),
            *_w2((512, 256)),
            pl.BlockSpec((1, 256), _whole),
            *_w2((256, 32)),
            pl.BlockSpec((1, 32), _whole),
        ],
        out_specs=pl.BlockSpec((bb, 32), _row_block),
        out_shape=jax.ShapeDtypeStruct((B, 32), jnp.float32),
    )


@functools.cache
def _make_top(bb: int):
    return pl.pallas_call(
        _top_body,
        grid=(B // bb,),
        in_specs=[
            pl.BlockSpec((bb, 32), _row_block),
            pl.BlockSpec((bb, F * D), _row_block),
            *_w2((32, 512)),
            *_w2((F * D, 512)),
            pl.BlockSpec((1, 512), _whole),
            *_w2((512, 256)),
            pl.BlockSpec((1, 256), _whole),
            *_w2((256, 1)),
            pl.BlockSpec((1, 1), _whole),
        ],
        out_specs=pl.BlockSpec((bb, 1), _row_block),
        out_shape=jax.ShapeDtypeStruct((B, 1), jnp.float32),
    )


def kernel(num, cat, bot_W0, bot_b0, bot_W1, bot_b1, bot_W2, bot_b2, tables,
           top_W0, top_b0, top_W1, top_b1, top_W2, top_b2):
    tab4 = _make_repack()(tables.transpose(0, 2, 1), jnp.eye(128, dtype=jnp.float32))
    c = cat.astype(jnp.int32)
    idx = (4 * (c % _V4) + c // _V4 +
           (jnp.arange(F, dtype=jnp.int32) * V)[None, :]).reshape(-1)
    emb_flat = _make_sc_gather()(tab4.reshape(F * V, D), idx).reshape(B, F * D)

    x32 = _make_bot(512)(
        num,
        *_split_w(bot_W0), bot_b0.reshape(1, -1),
        *_split_w(bot_W1), bot_b1.reshape(1, -1),
        *_split_w(bot_W2), bot_b2.reshape(1, -1),
    )
    bot = 32
    out = _make_top(512)(
        x32, emb_flat,
        *_split_w(top_W0[:bot]), *_split_w(top_W0[bot:]),
        top_b0.reshape(1, -1),
        *_split_w(top_W1), top_b1.reshape(1, -1),
        *_split_w(top_W2), top_b2.reshape(1, -1),
    )
    return out.reshape(B)
